# baseline (speedup 1.0000x reference)
import functools
import math

import jax
import jax.numpy as jnp
import numpy as np
from jax import lax
from jax.experimental import pallas as pl
from jax.experimental.pallas import tpu as pltpu
from jax.experimental.pallas import tpu_sc as plsc

D_MODEL = 1024
SEQ = 4096
NB = 4
ROWS = NB * SEQ
SCALE = math.sqrt(D_MODEL)

L = 16
NC, NS = 2, 16
NW = NC * NS
P_PER_W = SEQ // NW
CP = 32
NCHUNK = P_PER_W // CP


def _make_pe(seq_len: int, d_model: int) -> np.ndarray:
    position = np.arange(seq_len, dtype=np.float32)[:, None]
    div_term = np.exp(
        np.arange(0, d_model, 2, dtype=np.float32) * (-math.log(10000.0) / d_model)
    )
    pe = np.zeros((seq_len, d_model), dtype=np.float32)
    pe[:, 0::2] = np.sin(position * div_term)
    pe[:, 1::2] = np.cos(position * div_term)
    return pe


_PE = _make_pe(SEQ, D_MODEL)


def _sc_embed(x, lut, pe):
    mesh = plsc.VectorSubcoreMesh(core_axis_name="c", subcore_axis_name="s")

    NT = NCHUNK * NB

    @functools.partial(
        pl.kernel,
        mesh=mesh,
        out_type=jax.ShapeDtypeStruct((ROWS, D_MODEL), jnp.float32),
        scratch_types=[
            pltpu.VMEM((NB, P_PER_W), jnp.int32),
            pltpu.VMEM((CP, D_MODEL), jnp.float32),
            pltpu.VMEM((CP, D_MODEL), jnp.float32),
            pltpu.VMEM((CP, D_MODEL), jnp.float32),
            pltpu.SemaphoreType.DMA,
            pltpu.SemaphoreType.DMA,
            pltpu.SemaphoreType.DMA,
            pltpu.SemaphoreType.DMA,
            pltpu.SemaphoreType.DMA,
        ],
    )
    def k(x_hbm, lut_hbm, pe_hbm, out_hbm, idx_v, pe_v, row0, row1,
          g0, g1, s0, s1, psem):
        c = lax.axis_index("c")
        s = lax.axis_index("s")
        wid = s * NC + c
        pbase = wid * P_PER_W
        rows = (row0, row1)
        gsems = (g0, g1)
        ssems = (s0, s1)
        for b in range(NB):
            pltpu.sync_copy(x_hbm.at[b, pl.ds(pbase, P_PER_W)], idx_v.at[b])

        def gather(t):
            i = t % 2
            return pltpu.make_async_copy(
                lut_hbm.at[idx_v.at[t % NB, pl.ds((t // NB) * CP, CP)]],
                rows[i], gsems[i],
            )

        def store(t):
            i = t % 2
            off = (t // NB) * CP
            return pltpu.make_async_copy(
                rows[i],
                out_hbm.at[pl.ds((t % NB) * SEQ + pbase + off, CP)],
                ssems[i],
            )

        def pe_load(ci):
            return pltpu.make_async_copy(
                pe_hbm.at[pl.ds(pbase + ci * CP, CP)], pe_v, psem
            )

        gather(0).start()
        pe_load(0).start()
        gather(1).start()
        for t in range(NT):
            gather(t).wait()
            if t % NB == 0:
                pe_load(t // NB).wait()

            def rowloop(r, cr):
                rv = rows[t % 2]
                for v in range(D_MODEL // L):
                    sl = pl.ds(v * L, L)
                    rv[r, sl] = rv[r, sl] * SCALE + pe_v[r, sl]
                return cr

            lax.fori_loop(0, CP, rowloop, 0)
            if t % NB == NB - 1 and t + 1 < NT:
                pe_load(t // NB + 1).start()
            store(t).start()
            if t + 2 < NT:
                store(t).wait()
                gather(t + 2).start()
        store(NT - 2).wait()
        store(NT - 1).wait()

    return k(x, lut, pe)


def kernel(x, lut):
    pe = jnp.asarray(_PE)
    out = _sc_embed(x.astype(jnp.int32), lut, pe)
    return out.reshape(NB, SEQ, D_MODEL)
